# baseline (speedup 1.0000x reference)
import functools

import jax
import jax.numpy as jnp
from jax import lax
from jax.experimental import pallas as pl
from jax.experimental.pallas import tpu as pltpu
from jax.experimental.pallas import tpu_sc as plsc

_N = 10000
_NP = 10240
_E = 160000
_NT = 32
_RPT = _NP // _NT
_BBS = 8000
_NBLK = _E // _BBS
_ACC_R = 328
_F32 = jnp.float32
_I32 = jnp.int32


def _mesh():
    return plsc.VectorSubcoreMesh(core_axis_name="c", subcore_axis_name="s")


def _wid():
    return lax.axis_index("c") * 16 + lax.axis_index("s")


def _it16():
    return lax.iota(_I32, 16)


def _make_bin():
    lsz = _NT * _NBLK * _BBS

    @functools.partial(
        pl.kernel,
        out_type=[
            jax.ShapeDtypeStruct((lsz,), _I32),
            jax.ShapeDtypeStruct((lsz,), _I32),
            jax.ShapeDtypeStruct((lsz,), _I32),
            jax.ShapeDtypeStruct((_NT * 32,), _I32),
            jax.ShapeDtypeStruct((_NP, 16), _F32),
        ],
        mesh=_mesh(),
        compiler_params=pltpu.CompilerParams(needs_layout_passes=False),
        scratch_types=[
            pltpu.VMEM((_BBS,), _I32),
            pltpu.VMEM((_BBS,), _I32),
            pltpu.VMEM((_BBS,), _I32),
            pltpu.VMEM((_BBS,), _I32),
            pltpu.VMEM((_BBS,), _I32),
            pltpu.VMEM((32,), _I32),
            pltpu.VMEM((_RPT, 16), _F32),
        ],
    )
    def k(src_hbm, dst_hbm, z_hbm, eids, srcs, locs, cnts, deg16,
          dstbuf, srcbuf, eidb, srcb, locb, cntb, degacc):
        wid = _wid()
        lo = wid * _RPT
        it16 = _it16()
        ones16 = jnp.ones((16,), _F32)
        zero16i = jnp.zeros((16,), _I32)
        pltpu.sync_copy(z_hbm, degacc)
        plsc.store_scatter(cntb, [it16], zero16i)
        plsc.store_scatter(cntb, [16 + it16], zero16i)

        def blk_body(blk, _):
            eoff = blk * _BBS
            pltpu.sync_copy(dst_hbm.at[pl.ds(eoff, _BBS)], dstbuf)
            pltpu.sync_copy(src_hbm.at[pl.ds(eoff, _BBS)], srcbuf)

            def scan_body(i, ptr):
                v = plsc.load_gather(dstbuf, [i * 16 + it16])
                sv = plsc.load_gather(srcbuf, [i * 16 + it16])
                loc = v - lo
                m = (loc >= 0) & (loc < _RPT)
                plsc.addupdate_scatter(degacc, [loc, it16], ones16, mask=m)
                csum = plsc.cumsum(m.astype(_I32))
                pos = ptr + csum - 1
                plsc.store_scatter(eidb, [pos], eoff + i * 16 + it16, mask=m)
                plsc.store_scatter(srcb, [pos], sv, mask=m)
                plsc.store_scatter(locb, [pos], loc, mask=m)
                return ptr + jnp.max(csum)

            nm = lax.fori_loop(0, _BBS // 16, scan_body, 0)
            plsc.store_scatter(cntb, [jnp.full((16,), blk, _I32)],
                               jnp.full((16,), nm, _I32), mask=it16 == 0)
            base = (wid * _NBLK + blk) * _BBS
            pltpu.sync_copy(eidb, eids.at[pl.ds(base, _BBS)])
            pltpu.sync_copy(srcb, srcs.at[pl.ds(base, _BBS)])
            pltpu.sync_copy(locb, locs.at[pl.ds(base, _BBS)])
            return 0

        lax.fori_loop(0, _NBLK, blk_body, 0)
        pltpu.sync_copy(cntb, cnts.at[pl.ds(wid * 32, 32)])
        pltpu.sync_copy(degacc, deg16.at[pl.ds(lo, _RPT)])

    return k


def _make_reduce(nch, nrows, is_add):
    nck = nch // 16

    @functools.partial(
        pl.kernel,
        out_type=jax.ShapeDtypeStruct((_NP, nch), _F32),
        mesh=_mesh(),
        compiler_params=pltpu.CompilerParams(needs_layout_passes=False),
        scratch_types=[
            pltpu.VMEM((_BBS + 32,), _I32),
            pltpu.VMEM((_BBS + 32,), _I32),
            pltpu.VMEM((32,), _I32),
            pltpu.VMEM((16, nch), _F32),
            pltpu.VMEM((16, nch), _F32),
            pltpu.VMEM((_ACC_R, nch), _F32),
            pltpu.SemaphoreType.DMA,
            pltpu.SemaphoreType.DMA,
        ],
    )
    def k(glist_hbm, locs_hbm, cnts_hbm, table, z_hbm, out,
          glist, locb, cntv, rb0, rb1, acc, sem0, sem1):
        wid = _wid()
        lo = wid * _RPT
        it16 = _it16()
        pltpu.sync_copy(z_hbm, acc)
        pltpu.sync_copy(cnts_hbm.at[pl.ds(wid * 32, 32)], cntv)

        def rmw(rowbuf, locv, rbase):
            def jbody(j, _):
                rot = (it16 + j) & 15
                for c in range(nck):
                    cidx = c * 16 + rot
                    u = plsc.load_gather(rowbuf, [rbase + it16, cidx])
                    if is_add:
                        plsc.addupdate_scatter(acc, [locv, cidx], u)
                    else:
                        cur = plsc.load_gather(acc, [locv, cidx])
                        plsc.store_scatter(acc, [locv, cidx],
                                           jnp.maximum(cur, u))
                return 0

            lax.fori_loop(0, 16, jbody, 0)

        def blk_body(blk, _):
            base = (wid * _NBLK + blk) * _BBS
            pltpu.sync_copy(glist_hbm.at[pl.ds(base, _BBS)],
                            glist.at[pl.ds(0, _BBS)])
            pltpu.sync_copy(locs_hbm.at[pl.ds(base, _BBS)],
                            locb.at[pl.ds(0, _BBS)])
            cnt = jnp.max(plsc.load_gather(cntv, [jnp.full((16,), blk, _I32)]))
            plsc.store_scatter(glist, [cnt + it16], jnp.zeros((16,), _I32))
            plsc.store_scatter(locb, [cnt + it16],
                               jnp.full((16,), _RPT, _I32))
            nb = (cnt + 15) // 16

            @pl.when(nb > 0)
            def _():
                g0 = plsc.load_gather(glist, [it16])
                pltpu.async_copy(table.at[g0], rb0, sem0)

            def proc(b, _):
                @pl.when(b + 1 < nb)
                def _():
                    gn = plsc.load_gather(glist, [(b + 1) * 16 + it16])

                    @pl.when(lax.rem(b + 1, 2) == 0)
                    def _():
                        pltpu.async_copy(table.at[gn], rb0, sem0)

                    @pl.when(lax.rem(b + 1, 2) == 1)
                    def _():
                        pltpu.async_copy(table.at[gn], rb1, sem1)

                locv = plsc.load_gather(locb, [b * 16 + it16])

                @pl.when(lax.rem(b, 2) == 0)
                def _():
                    pltpu.make_async_copy(table.at[pl.ds(0, 16)], rb0,
                                          sem0).wait()
                    rmw(rb0, locv, 0)

                @pl.when(lax.rem(b, 2) == 1)
                def _():
                    pltpu.make_async_copy(table.at[pl.ds(0, 16)], rb1,
                                          sem1).wait()
                    rmw(rb1, locv, 0)

                return 0

            lax.fori_loop(0, nb, proc, 0)
            return 0

        lax.fori_loop(0, _NBLK, blk_body, 0)
        pltpu.sync_copy(acc.at[pl.ds(0, _RPT)], out.at[pl.ds(lo, _RPT)])

    return k


def _make_edge_gather(bs):
    ept = _E // _NT
    nblk = ept // bs

    @functools.partial(
        pl.kernel,
        out_type=[jax.ShapeDtypeStruct((_E, 512), _F32)] * 2,
        mesh=_mesh(),
        compiler_params=pltpu.CompilerParams(needs_layout_passes=False),
        scratch_types=[
            pltpu.VMEM((bs,), _I32),
            pltpu.VMEM((bs,), _I32),
            pltpu.VMEM((bs, 512), _F32),
            pltpu.SemaphoreType.DMA,
        ],
    )
    def k(src_hbm, dst_hbm, p_hbm, q_hbm, pg, qg, idxd, idxs, rb, sem):
        base = _wid() * ept

        def blk_body(blk, _):
            off = base + blk * bs
            pltpu.sync_copy(dst_hbm.at[pl.ds(off, bs)], idxd)
            pltpu.sync_copy(src_hbm.at[pl.ds(off, bs)], idxs)
            pltpu.async_copy(p_hbm.at[idxd], rb, sem).wait()
            pltpu.sync_copy(rb, pg.at[pl.ds(off, bs)])
            pltpu.async_copy(q_hbm.at[idxs], rb, sem).wait()
            pltpu.sync_copy(rb, qg.at[pl.ds(off, bs)])
            return 0

        lax.fori_loop(0, nblk, blk_body, 0)

    return k


def _tc1_body(deg_ref, x_ref, dis_ref, gx_ref):
    deg = jnp.sum(deg_ref[...], axis=1, keepdims=True) + 1.0
    dis = lax.rsqrt(deg)
    dis_ref[...] = jnp.broadcast_to(dis, dis_ref.shape)
    gx_ref[...] = x_ref[...] * dis


def _tc1(deg16, x):
    nb = 10
    blk = _N // nb
    return pl.pallas_call(
        _tc1_body,
        grid=(nb,),
        in_specs=[
            pl.BlockSpec((blk, 16), lambda i: (i, 0)),
            pl.BlockSpec((blk, 256), lambda i: (i, 0)),
        ],
        out_specs=[
            pl.BlockSpec((blk, 16), lambda i: (i, 0)),
            pl.BlockSpec((blk, 256), lambda i: (i, 0)),
        ],
        out_shape=[
            jax.ShapeDtypeStruct((_N, 16), _F32),
            jax.ShapeDtypeStruct((_N, 256), _F32),
        ],
    )(deg16, x)


def _dotb(a, b):
    return jnp.dot(a.astype(jnp.bfloat16), b.astype(jnp.bfloat16),
                   preferred_element_type=_F32)


def _tc2_body(sx_ref, gx_ref, dis_ref, b1_ref, w1_ref, w2_ref, g2_ref):
    dis = dis_ref[..., :1]
    ax = (sx_ref[...] + gx_ref[...]) * dis
    h1 = jnp.maximum(_dotb(ax, w1_ref[...]) + b1_ref[...], 0.0)
    g2_ref[...] = _dotb(h1, w2_ref[...]) * dis


def _tc2(sx, gx, dis16, b1, w1, w2):
    nb = 10
    blk = _N // nb
    io = lambda i: (i, 0)
    w = lambda i: (0, 0)
    return pl.pallas_call(
        _tc2_body,
        grid=(nb,),
        in_specs=[
            pl.BlockSpec((blk, 256), io), pl.BlockSpec((blk, 256), io),
            pl.BlockSpec((blk, 16), io),
            pl.BlockSpec((1, 512), w),
            pl.BlockSpec((256, 512), w),
            pl.BlockSpec((512, 128), w),
        ],
        out_specs=pl.BlockSpec((blk, 128), io),
        out_shape=jax.ShapeDtypeStruct((_N, 128), _F32),
    )(sx, gx, dis16, b1, w1, w2)


def _tc3_body(s2_ref, g2_ref, dis_ref, b2_ref, wm1_ref, bm1_ref, p_ref,
              q_ref):
    dis = dis_ref[..., :1]
    h2 = (s2_ref[...] + g2_ref[...]) * dis + b2_ref[...]
    a = wm1_ref[:128, :]
    b = wm1_ref[128:, :]
    p_ref[...] = _dotb(h2, a - b) + bm1_ref[...]
    q_ref[...] = _dotb(h2, b)


def _tc3(s2, g2, dis16, b2, wm1, bm1):
    nb = 10
    blk = _N // nb
    io = lambda i: (i, 0)
    w = lambda i: (0, 0)
    return pl.pallas_call(
        _tc3_body,
        grid=(nb,),
        in_specs=[
            pl.BlockSpec((blk, 128), io), pl.BlockSpec((blk, 128), io),
            pl.BlockSpec((blk, 16), io),
            pl.BlockSpec((1, 128), w),
            pl.BlockSpec((256, 512), w),
            pl.BlockSpec((1, 512), w),
        ],
        out_specs=[pl.BlockSpec((blk, 512), io)] * 2,
        out_shape=[jax.ShapeDtypeStruct((_N, 512), _F32)] * 2,
    )(s2, g2, dis16, b2, wm1, bm1)


def _tc4_body(pg_ref, qg_ref, wm2_ref, bm2_ref, u0_ref, u1_ref):
    t = jnp.maximum(pg_ref[...] + qg_ref[...], 0.0)
    u = _dotb(t, wm2_ref[...]) + bm2_ref[...]
    u0_ref[...] = u[:, :256]
    u1_ref[...] = u[:, 256:]


def _tc4(pg, qg, wm2, bm2):
    nb = 250
    blk = _E // nb
    io = lambda i: (i, 0)
    w = lambda i: (0, 0)
    return pl.pallas_call(
        _tc4_body,
        grid=(nb,),
        in_specs=[
            pl.BlockSpec((blk, 512), io), pl.BlockSpec((blk, 512), io),
            pl.BlockSpec((512, 512), w), pl.BlockSpec((1, 512), w),
        ],
        out_specs=[pl.BlockSpec((blk, 256), io)] * 2,
        out_shape=[jax.ShapeDtypeStruct((_E, 256), _F32)] * 2,
    )(pg, qg, wm2, bm2)


def _tc5_body(h30_ref, h31_ref, wf_ref, bf_ref, o_ref):
    h3 = jnp.concatenate([h30_ref[...], h31_ref[...]], axis=1)
    o_ref[...] = jnp.maximum(_dotb(h3, wf_ref[...]) + bf_ref[...], 0.0)


def _tc5(h30, h31, wf, bf):
    nb = 10
    blk = _N // nb
    return pl.pallas_call(
        _tc5_body,
        grid=(nb,),
        in_specs=[
            pl.BlockSpec((blk, 256), lambda i: (i, 0)),
            pl.BlockSpec((blk, 256), lambda i: (i, 0)),
            pl.BlockSpec((512, 2), lambda i: (0, 0)),
            pl.BlockSpec((1, 2), lambda i: (0, 0)),
        ],
        out_specs=pl.BlockSpec((blk, 2), lambda i: (i, 0)),
        out_shape=jax.ShapeDtypeStruct((_N, 2), _F32),
    )(h30, h31, wf, bf)


_bin_k = _make_bin()
_agg256_k = _make_reduce(256, _RPT, True)
_agg128_k = _make_reduce(128, _RPT, True)
_smax_k = _make_reduce(256, _RPT, False)
_egather_k = _make_edge_gather(200)


def kernel(x, edge_index, W1, b1, W2, b2, Wm1, bm1, Wm2, bm2, Wf, bf):
    src = edge_index[0].astype(_I32)
    dst = edge_index[1].astype(_I32)

    z16 = jnp.zeros((_RPT, 16), _F32)
    z256 = jnp.zeros((_ACC_R, 256), _F32)
    z128 = jnp.zeros((_ACC_R, 128), _F32)

    eids, srcs, locs, cnts, deg16 = _bin_k(src, dst, z16)
    dis16, gx = _tc1(deg16[:_N], x)
    sx = _agg256_k(srcs, locs, cnts, gx, z256)
    g2 = _tc2(sx[:_N], gx, dis16, b1.reshape(1, 512), W1, W2)
    s2 = _agg128_k(srcs, locs, cnts, g2, z128)
    p, q = _tc3(s2[:_N], g2, dis16, b2.reshape(1, 128), Wm1,
                bm1.reshape(1, 512))
    pg, qg = _egather_k(src, dst, p, q)
    u0, u1 = _tc4(pg, qg, Wm2, bm2.reshape(1, 512))
    h30 = _smax_k(eids, locs, cnts, u0, z256)
    h31 = _smax_k(eids, locs, cnts, u1, z256)
    return _tc5(h30[:_N], h31[:_N], Wf, bf.reshape(1, 2))


# speedup vs baseline: 1.0306x; 1.0306x over previous
import functools

import jax
import jax.numpy as jnp
from jax import lax
from jax.experimental import pallas as pl
from jax.experimental.pallas import tpu as pltpu
from jax.experimental.pallas import tpu_sc as plsc

_N = 10000
_NP = 10240
_E = 160000
_NT = 32
_RPT = _NP // _NT
_BBS = 8000
_NBLK = _E // _BBS
_ACC_R = 328
_F32 = jnp.float32
_I32 = jnp.int32


def _mesh():
    return plsc.VectorSubcoreMesh(core_axis_name="c", subcore_axis_name="s")


def _wid():
    return lax.axis_index("c") * 16 + lax.axis_index("s")


def _it16():
    return lax.iota(_I32, 16)


def _make_bin():
    lsz = _NT * _NBLK * _BBS

    @functools.partial(
        pl.kernel,
        out_type=[
            jax.ShapeDtypeStruct((lsz,), _I32),
            jax.ShapeDtypeStruct((lsz,), _I32),
            jax.ShapeDtypeStruct((lsz,), _I32),
            jax.ShapeDtypeStruct((_NT * 32,), _I32),
            jax.ShapeDtypeStruct((_NP, 16), _F32),
        ],
        mesh=_mesh(),
        compiler_params=pltpu.CompilerParams(needs_layout_passes=False),
        scratch_types=[
            pltpu.VMEM((_BBS,), _I32),
            pltpu.VMEM((_BBS,), _I32),
            pltpu.VMEM((_BBS,), _I32),
            pltpu.VMEM((_BBS,), _I32),
            pltpu.VMEM((_BBS,), _I32),
            pltpu.VMEM((32,), _I32),
            pltpu.VMEM((_RPT, 16), _F32),
        ],
    )
    def k(src_hbm, dst_hbm, z_hbm, eids, srcs, locs, cnts, deg16,
          dstbuf, srcbuf, eidb, srcb, locb, cntb, degacc):
        wid = _wid()
        lo = wid * _RPT
        it16 = _it16()
        ones16 = jnp.ones((16,), _F32)
        zero16i = jnp.zeros((16,), _I32)
        pltpu.sync_copy(z_hbm, degacc)
        plsc.store_scatter(cntb, [it16], zero16i)
        plsc.store_scatter(cntb, [16 + it16], zero16i)

        def blk_body(blk, _):
            eoff = blk * _BBS
            pltpu.sync_copy(dst_hbm.at[pl.ds(eoff, _BBS)], dstbuf)
            pltpu.sync_copy(src_hbm.at[pl.ds(eoff, _BBS)], srcbuf)

            def scan_body(i, ptr):
                v = plsc.load_gather(dstbuf, [i * 16 + it16])
                sv = plsc.load_gather(srcbuf, [i * 16 + it16])
                loc = v - lo
                m = (loc >= 0) & (loc < _RPT)
                plsc.addupdate_scatter(degacc, [loc, it16], ones16, mask=m)
                csum = plsc.cumsum(m.astype(_I32))
                pos = ptr + csum - 1
                plsc.store_scatter(eidb, [pos], eoff + i * 16 + it16, mask=m)
                plsc.store_scatter(srcb, [pos], sv, mask=m)
                plsc.store_scatter(locb, [pos], loc, mask=m)
                return ptr + jnp.max(csum)

            nm = lax.fori_loop(0, _BBS // 16, scan_body, 0)
            plsc.store_scatter(cntb, [jnp.full((16,), blk, _I32)],
                               jnp.full((16,), nm, _I32), mask=it16 == 0)
            base = (wid * _NBLK + blk) * _BBS
            pltpu.sync_copy(eidb, eids.at[pl.ds(base, _BBS)])
            pltpu.sync_copy(srcb, srcs.at[pl.ds(base, _BBS)])
            pltpu.sync_copy(locb, locs.at[pl.ds(base, _BBS)])
            return 0

        lax.fori_loop(0, _NBLK, blk_body, 0)
        pltpu.sync_copy(cntb, cnts.at[pl.ds(wid * 32, 32)])
        pltpu.sync_copy(degacc, deg16.at[pl.ds(lo, _RPT)])

    return k


def _make_reduce(nch, nrows, is_add):
    nck = nch // 16

    @functools.partial(
        pl.kernel,
        out_type=jax.ShapeDtypeStruct((_NP, nch), _F32),
        mesh=_mesh(),
        compiler_params=pltpu.CompilerParams(needs_layout_passes=False),
        scratch_types=[
            pltpu.VMEM((_BBS + 32,), _I32),
            pltpu.VMEM((_BBS + 32,), _I32),
            pltpu.VMEM((32,), _I32),
            pltpu.VMEM((16, nch), _F32),
            pltpu.VMEM((16, nch), _F32),
            pltpu.VMEM((_ACC_R, nch), _F32),
            pltpu.SemaphoreType.DMA,
            pltpu.SemaphoreType.DMA,
        ],
    )
    def k(glist_hbm, locs_hbm, cnts_hbm, table, z_hbm, out,
          glist, locb, cntv, rb0, rb1, acc, sem0, sem1):
        wid = _wid()
        lo = wid * _RPT
        it16 = _it16()
        pltpu.sync_copy(z_hbm, acc)
        pltpu.sync_copy(cnts_hbm.at[pl.ds(wid * 32, 32)], cntv)

        def rmw(rowbuf, locv, rbase):
            def jbody(j4, _):
                for dj in range(4):
                    rot = (it16 + (j4 * 4 + dj)) & 15
                    for c in range(nck):
                        cidx = c * 16 + rot
                        u = plsc.load_gather(rowbuf, [rbase + it16, cidx])
                        if is_add:
                            plsc.addupdate_scatter(acc, [locv, cidx], u)
                        else:
                            cur = plsc.load_gather(acc, [locv, cidx])
                            plsc.store_scatter(acc, [locv, cidx],
                                               jnp.maximum(cur, u))
                return 0

            lax.fori_loop(0, 4, jbody, 0)

        def blk_body(blk, _):
            base = (wid * _NBLK + blk) * _BBS
            pltpu.sync_copy(glist_hbm.at[pl.ds(base, _BBS)],
                            glist.at[pl.ds(0, _BBS)])
            pltpu.sync_copy(locs_hbm.at[pl.ds(base, _BBS)],
                            locb.at[pl.ds(0, _BBS)])
            cnt = jnp.max(plsc.load_gather(cntv, [jnp.full((16,), blk, _I32)]))
            plsc.store_scatter(glist, [cnt + it16], jnp.zeros((16,), _I32))
            plsc.store_scatter(locb, [cnt + it16],
                               jnp.full((16,), _RPT, _I32))
            nb = (cnt + 15) // 16

            @pl.when(nb > 0)
            def _():
                g0 = plsc.load_gather(glist, [it16])
                pltpu.async_copy(table.at[g0], rb0, sem0)

            def proc(b, _):
                @pl.when(b + 1 < nb)
                def _():
                    gn = plsc.load_gather(glist, [(b + 1) * 16 + it16])

                    @pl.when(lax.rem(b + 1, 2) == 0)
                    def _():
                        pltpu.async_copy(table.at[gn], rb0, sem0)

                    @pl.when(lax.rem(b + 1, 2) == 1)
                    def _():
                        pltpu.async_copy(table.at[gn], rb1, sem1)

                locv = plsc.load_gather(locb, [b * 16 + it16])

                @pl.when(lax.rem(b, 2) == 0)
                def _():
                    pltpu.make_async_copy(table.at[pl.ds(0, 16)], rb0,
                                          sem0).wait()
                    rmw(rb0, locv, 0)

                @pl.when(lax.rem(b, 2) == 1)
                def _():
                    pltpu.make_async_copy(table.at[pl.ds(0, 16)], rb1,
                                          sem1).wait()
                    rmw(rb1, locv, 0)

                return 0

            lax.fori_loop(0, nb, proc, 0)
            return 0

        lax.fori_loop(0, _NBLK, blk_body, 0)
        pltpu.sync_copy(acc.at[pl.ds(0, _RPT)], out.at[pl.ds(lo, _RPT)])

    return k


def _make_edge_gather(bs):
    ept = _E // _NT
    nblk = ept // bs

    @functools.partial(
        pl.kernel,
        out_type=[jax.ShapeDtypeStruct((_E, 512), _F32)] * 2,
        mesh=_mesh(),
        compiler_params=pltpu.CompilerParams(needs_layout_passes=False),
        scratch_types=[
            pltpu.VMEM((bs,), _I32),
            pltpu.VMEM((bs,), _I32),
            pltpu.VMEM((bs, 512), _F32),
            pltpu.SemaphoreType.DMA,
        ],
    )
    def k(src_hbm, dst_hbm, p_hbm, q_hbm, pg, qg, idxd, idxs, rb, sem):
        base = _wid() * ept

        def blk_body(blk, _):
            off = base + blk * bs
            pltpu.sync_copy(dst_hbm.at[pl.ds(off, bs)], idxd)
            pltpu.sync_copy(src_hbm.at[pl.ds(off, bs)], idxs)
            pltpu.async_copy(p_hbm.at[idxd], rb, sem).wait()
            pltpu.sync_copy(rb, pg.at[pl.ds(off, bs)])
            pltpu.async_copy(q_hbm.at[idxs], rb, sem).wait()
            pltpu.sync_copy(rb, qg.at[pl.ds(off, bs)])
            return 0

        lax.fori_loop(0, nblk, blk_body, 0)

    return k


def _tc1_body(deg_ref, x_ref, dis_ref, gx_ref):
    deg = jnp.sum(deg_ref[...], axis=1, keepdims=True) + 1.0
    dis = lax.rsqrt(deg)
    dis_ref[...] = jnp.broadcast_to(dis, dis_ref.shape)
    gx_ref[...] = x_ref[...] * dis


def _tc1(deg16, x):
    nb = 10
    blk = _N // nb
    return pl.pallas_call(
        _tc1_body,
        grid=(nb,),
        in_specs=[
            pl.BlockSpec((blk, 16), lambda i: (i, 0)),
            pl.BlockSpec((blk, 256), lambda i: (i, 0)),
        ],
        out_specs=[
            pl.BlockSpec((blk, 16), lambda i: (i, 0)),
            pl.BlockSpec((blk, 256), lambda i: (i, 0)),
        ],
        out_shape=[
            jax.ShapeDtypeStruct((_N, 16), _F32),
            jax.ShapeDtypeStruct((_N, 256), _F32),
        ],
    )(deg16, x)


def _dotb(a, b):
    return jnp.dot(a.astype(jnp.bfloat16), b.astype(jnp.bfloat16),
                   preferred_element_type=_F32)


def _tc2_body(sx_ref, gx_ref, dis_ref, b1_ref, w1_ref, w2_ref, g2_ref):
    dis = dis_ref[..., :1]
    ax = (sx_ref[...] + gx_ref[...]) * dis
    h1 = jnp.maximum(_dotb(ax, w1_ref[...]) + b1_ref[...], 0.0)
    g2_ref[...] = _dotb(h1, w2_ref[...]) * dis


def _tc2(sx, gx, dis16, b1, w1, w2):
    nb = 10
    blk = _N // nb
    io = lambda i: (i, 0)
    w = lambda i: (0, 0)
    return pl.pallas_call(
        _tc2_body,
        grid=(nb,),
        in_specs=[
            pl.BlockSpec((blk, 256), io), pl.BlockSpec((blk, 256), io),
            pl.BlockSpec((blk, 16), io),
            pl.BlockSpec((1, 512), w),
            pl.BlockSpec((256, 512), w),
            pl.BlockSpec((512, 128), w),
        ],
        out_specs=pl.BlockSpec((blk, 128), io),
        out_shape=jax.ShapeDtypeStruct((_N, 128), _F32),
    )(sx, gx, dis16, b1, w1, w2)


def _tc3_body(s2_ref, g2_ref, dis_ref, b2_ref, wm1_ref, bm1_ref, p_ref,
              q_ref):
    dis = dis_ref[..., :1]
    h2 = (s2_ref[...] + g2_ref[...]) * dis + b2_ref[...]
    a = wm1_ref[:128, :]
    b = wm1_ref[128:, :]
    p_ref[...] = _dotb(h2, a - b) + bm1_ref[...]
    q_ref[...] = _dotb(h2, b)


def _tc3(s2, g2, dis16, b2, wm1, bm1):
    nb = 10
    blk = _N // nb
    io = lambda i: (i, 0)
    w = lambda i: (0, 0)
    return pl.pallas_call(
        _tc3_body,
        grid=(nb,),
        in_specs=[
            pl.BlockSpec((blk, 128), io), pl.BlockSpec((blk, 128), io),
            pl.BlockSpec((blk, 16), io),
            pl.BlockSpec((1, 128), w),
            pl.BlockSpec((256, 512), w),
            pl.BlockSpec((1, 512), w),
        ],
        out_specs=[pl.BlockSpec((blk, 512), io)] * 2,
        out_shape=[jax.ShapeDtypeStruct((_N, 512), _F32)] * 2,
    )(s2, g2, dis16, b2, wm1, bm1)


def _tc4_body(pg_ref, qg_ref, wm2_ref, bm2_ref, u0_ref, u1_ref):
    t = jnp.maximum(pg_ref[...] + qg_ref[...], 0.0)
    u = _dotb(t, wm2_ref[...]) + bm2_ref[...]
    u0_ref[...] = u[:, :256]
    u1_ref[...] = u[:, 256:]


def _tc4(pg, qg, wm2, bm2):
    nb = 250
    blk = _E // nb
    io = lambda i: (i, 0)
    w = lambda i: (0, 0)
    return pl.pallas_call(
        _tc4_body,
        grid=(nb,),
        in_specs=[
            pl.BlockSpec((blk, 512), io), pl.BlockSpec((blk, 512), io),
            pl.BlockSpec((512, 512), w), pl.BlockSpec((1, 512), w),
        ],
        out_specs=[pl.BlockSpec((blk, 256), io)] * 2,
        out_shape=[jax.ShapeDtypeStruct((_E, 256), _F32)] * 2,
    )(pg, qg, wm2, bm2)


def _tc5_body(h30_ref, h31_ref, wf_ref, bf_ref, o_ref):
    h3 = jnp.concatenate([h30_ref[...], h31_ref[...]], axis=1)
    o_ref[...] = jnp.maximum(_dotb(h3, wf_ref[...]) + bf_ref[...], 0.0)


def _tc5(h30, h31, wf, bf):
    nb = 10
    blk = _N // nb
    return pl.pallas_call(
        _tc5_body,
        grid=(nb,),
        in_specs=[
            pl.BlockSpec((blk, 256), lambda i: (i, 0)),
            pl.BlockSpec((blk, 256), lambda i: (i, 0)),
            pl.BlockSpec((512, 2), lambda i: (0, 0)),
            pl.BlockSpec((1, 2), lambda i: (0, 0)),
        ],
        out_specs=pl.BlockSpec((blk, 2), lambda i: (i, 0)),
        out_shape=jax.ShapeDtypeStruct((_N, 2), _F32),
    )(h30, h31, wf, bf)


_bin_k = _make_bin()
_agg256_k = _make_reduce(256, _RPT, True)
_agg128_k = _make_reduce(128, _RPT, True)
_smax_k = _make_reduce(256, _RPT, False)
_egather_k = _make_edge_gather(200)


def kernel(x, edge_index, W1, b1, W2, b2, Wm1, bm1, Wm2, bm2, Wf, bf):
    src = edge_index[0].astype(_I32)
    dst = edge_index[1].astype(_I32)

    z16 = jnp.zeros((_RPT, 16), _F32)
    z256 = jnp.zeros((_ACC_R, 256), _F32)
    z128 = jnp.zeros((_ACC_R, 128), _F32)

    eids, srcs, locs, cnts, deg16 = _bin_k(src, dst, z16)
    dis16, gx = _tc1(deg16[:_N], x)
    sx = _agg256_k(srcs, locs, cnts, gx, z256)
    g2 = _tc2(sx[:_N], gx, dis16, b1.reshape(1, 512), W1, W2)
    s2 = _agg128_k(srcs, locs, cnts, g2, z128)
    p, q = _tc3(s2[:_N], g2, dis16, b2.reshape(1, 128), Wm1,
                bm1.reshape(1, 512))
    pg, qg = _egather_k(src, dst, p, q)
    u0, u1 = _tc4(pg, qg, Wm2, bm2.reshape(1, 512))
    h30 = _smax_k(eids, locs, cnts, u0, z256)
    h31 = _smax_k(eids, locs, cnts, u1, z256)
    return _tc5(h30[:_N], h31[:_N], Wf, bf.reshape(1, 2))
